# baseline (speedup 1.0000x reference)
import functools

import jax
import jax.numpy as jnp
from jax import lax
from jax.experimental import pallas as pl
from jax.experimental.pallas import tpu as pltpu
from jax.experimental.pallas import tpu_sc as plsc

_T = 2048
_H = 1024
_F = 4096
_E = 16
_B = 256
_NT = _T // _B
_NBR = _NT + _E
_NSTEP = _NBR + _NT
_FC = 512
_NF = _F // _FC
_NW = 32
_BPW = _T // _NW


def _sigmoid(x):
    return 1.0 / (1.0 + jnp.exp(-x))


def _router_body(x_ref, wr_ref, idx_ref, w_ref):
    logits = lax.dot_general(x_ref[...], wr_ref[...], (((1,), (1,)), ((), ())),
                             preferred_element_type=jnp.float32)
    top = jnp.max(logits, axis=1, keepdims=True)
    idx_ref[...] = jnp.argmax(logits, axis=1).astype(jnp.int32)[:, None]
    w_ref[...] = _sigmoid(top)


def _router(x, Wr):
    return pl.pallas_call(
        _router_body,
        out_shape=(jax.ShapeDtypeStruct((_T, 1), jnp.int32),
                   jax.ShapeDtypeStruct((_T, 1), jnp.float32)),
    )(x, Wr)


def _route_metadata(idx):
    counts = jnp.zeros((_E,), jnp.int32).at[idx].add(1)
    ends_c = jnp.cumsum(counts)
    offs = ends_c - counts
    tile_lo = offs // _B
    tile_hi = (ends_c + _B - 1) // _B
    nt = jnp.where(counts > 0, tile_hi - tile_lo, 0)
    cum_nt = jnp.cumsum(nt)
    total = cum_nt[_E - 1]
    s = jnp.arange(_NBR, dtype=jnp.int32)
    g = jnp.searchsorted(cum_nt, s, side="right").astype(jnp.int32)
    g = jnp.minimum(g, _E - 1)
    prev = jnp.where(g > 0, cum_nt[jnp.maximum(g - 1, 0)], 0)
    k = s - prev
    tile = tile_lo[g] + k
    start = jnp.maximum(offs[g], tile * _B)
    end = jnp.minimum(ends_c[g], (tile + 1) * _B)
    valid = s < total
    last = jnp.maximum(total - 1, 0)
    g = jnp.where(valid, g, g[last])
    tile = jnp.clip(jnp.where(valid, tile, tile[last]), 0, _NT - 1)
    start = jnp.where(valid, start, 0)
    end = jnp.where(valid, end, 0)
    stile = jnp.arange(_NT, dtype=jnp.int32)
    tmap = jnp.concatenate([tile, stile]).astype(jnp.int32)
    gmap = jnp.concatenate([g, jnp.zeros((_NT,), jnp.int32)]).astype(jnp.int32)
    smap = jnp.concatenate([start, stile * _B]).astype(jnp.int32)
    emap = jnp.concatenate([end, (stile + 1) * _B]).astype(jnp.int32)
    return tmap, gmap, smap, emap


def _gather_body(x_hbm, order_hbm, w_hbm, xs_hbm, ws_hbm,
                 idx_v, rows_v, wsv_v, sem, sem2):
    wid = lax.axis_index("s") * 2 + lax.axis_index("c")
    base = wid * _BPW
    pltpu.sync_copy(order_hbm.at[pl.ds(base, _BPW)], idx_v)
    cp_x = pltpu.async_copy(x_hbm.at[idx_v], rows_v, sem)
    cp_w = pltpu.async_copy(w_hbm.at[idx_v], wsv_v, sem2)
    cp_x.wait()
    cp_w.wait()
    pltpu.sync_copy(rows_v, xs_hbm.at[pl.ds(base, _BPW)])
    pltpu.sync_copy(wsv_v, ws_hbm.at[pl.ds(base, _BPW)])


def _sc_gather(x, order, w128):
    return pl.kernel(
        _gather_body,
        out_type=(jax.ShapeDtypeStruct((_T, _H), jnp.float32),
                  jax.ShapeDtypeStruct((_T, 128), jnp.float32)),
        mesh=plsc.VectorSubcoreMesh(core_axis_name="c", subcore_axis_name="s"),
        scratch_types=[pltpu.VMEM((_BPW,), jnp.int32),
                       pltpu.VMEM((_BPW, _H), jnp.float32),
                       pltpu.VMEM((_BPW, 128), jnp.float32),
                       pltpu.SemaphoreType.DMA,
                       pltpu.SemaphoreType.DMA],
    )(x, order, w128)


def _scatter_body(osort_hbm, order_hbm, out_hbm, idx_v, rows_v, sem):
    wid = lax.axis_index("s") * 2 + lax.axis_index("c")
    base = wid * _BPW
    pltpu.sync_copy(order_hbm.at[pl.ds(base, _BPW)], idx_v)
    pltpu.sync_copy(osort_hbm.at[pl.ds(base, _BPW)], rows_v)
    pltpu.async_copy(rows_v, out_hbm.at[idx_v], sem).wait()


def _sc_scatter(osort, order):
    return pl.kernel(
        _scatter_body,
        out_type=jax.ShapeDtypeStruct((_T, _H), jnp.float32),
        mesh=plsc.VectorSubcoreMesh(core_axis_name="c", subcore_axis_name="s"),
        scratch_types=[pltpu.VMEM((_BPW,), jnp.int32),
                       pltpu.VMEM((_BPW, _H), jnp.float32),
                       pltpu.SemaphoreType.DMA],
    )(osort, order)


def _moe_body(tmap, gmap, smap, emap, xs_ref, ws_ref,
              w1_ref, v1_ref, w2_ref, gw_ref, uw_ref, dw_ref, out_ref):
    f = pl.program_id(0)
    j = pl.program_id(1)

    @pl.when((f == 0) & (j == 0))
    def _init():
        out_ref[...] = jnp.zeros_like(out_ref)

    t = tmap[j]
    st = smap[j]
    en = emap[j]

    @pl.when((j < _NBR) & (st < en))
    def _routed():
        xq = xs_ref[pl.ds(t * _B, _B), :].astype(jnp.bfloat16)
        a = lax.dot_general(xq, w1_ref[0].astype(jnp.bfloat16),
                            (((1,), (1,)), ((), ())),
                            preferred_element_type=jnp.float32)
        b = lax.dot_general(xq, v1_ref[0].astype(jnp.bfloat16),
                            (((1,), (1,)), ((), ())),
                            preferred_element_type=jnp.float32)
        h = a * _sigmoid(a) * b
        rows = t * _B + lax.broadcasted_iota(jnp.int32, (_B, 1), 0)
        h = jnp.where((rows >= st) & (rows < en), h * ws_ref[:, 0:1], 0.0)
        out_ref[pl.ds(t * _B, _B), :] += lax.dot_general(
            h.astype(jnp.bfloat16), w2_ref[0].astype(jnp.bfloat16),
            (((1,), (0,)), ((), ())),
            preferred_element_type=jnp.float32)

    @pl.when(j >= _NBR)
    def _shared():
        xq = xs_ref[pl.ds(t * _B, _B), :].astype(jnp.bfloat16)
        a = lax.dot_general(xq, gw_ref[...].astype(jnp.bfloat16),
                            (((1,), (1,)), ((), ())),
                            preferred_element_type=jnp.float32)
        b = lax.dot_general(xq, uw_ref[...].astype(jnp.bfloat16),
                            (((1,), (1,)), ((), ())),
                            preferred_element_type=jnp.float32)
        h = a * _sigmoid(a) * b
        out_ref[pl.ds(t * _B, _B), :] += lax.dot_general(
            h.astype(jnp.bfloat16), dw_ref[...].astype(jnp.bfloat16),
            (((1,), (1,)), ((), ())),
            preferred_element_type=jnp.float32)


def _moe_grid_spec():
    return pltpu.PrefetchScalarGridSpec(
        num_scalar_prefetch=4,
        grid=(_NF, _NSTEP),
        in_specs=[
            pl.BlockSpec((_T, _H), lambda f, j, tm, gm, sm, em: (0, 0)),
            pl.BlockSpec((_B, 128), lambda f, j, tm, gm, sm, em: (tm[j], 0)),
            pl.BlockSpec((1, _FC, _H), lambda f, j, tm, gm, sm, em: (gm[j], f, 0)),
            pl.BlockSpec((1, _FC, _H), lambda f, j, tm, gm, sm, em: (gm[j], f, 0)),
            pl.BlockSpec((1, _FC, _H), lambda f, j, tm, gm, sm, em: (gm[j], f, 0)),
            pl.BlockSpec((_FC, _H), lambda f, j, tm, gm, sm, em: (f, 0)),
            pl.BlockSpec((_FC, _H), lambda f, j, tm, gm, sm, em: (f, 0)),
            pl.BlockSpec((_H, _FC), lambda f, j, tm, gm, sm, em: (0, f)),
        ],
        out_specs=pl.BlockSpec((_T, _H), lambda f, j, tm, gm, sm, em: (0, 0)),
    )


def _moe_call(tmap, gmap, smap, emap, xs, ws, w1, v1, w2, gate_w, up_w, down_w):
    return pl.pallas_call(
        _moe_body,
        grid_spec=_moe_grid_spec(),
        out_shape=jax.ShapeDtypeStruct((_T, _H), jnp.float32),
        compiler_params=pltpu.CompilerParams(
            dimension_semantics=("arbitrary", "arbitrary")),
    )(tmap, gmap, smap, emap, xs, ws, w1, v1, w2, gate_w, up_w, down_w)


def kernel(hidden_states, Wr, w1, v1, w2, gate_w, up_w, down_w):
    x = hidden_states.reshape(_T, _H)
    idx2, w2d = _router(x, Wr)
    idx = idx2[:, 0]
    order = jnp.argsort(idx, stable=True).astype(jnp.int32)
    tmap, gmap, smap, emap = _route_metadata(idx)
    w128 = jnp.broadcast_to(w2d, (_T, 128))
    xs, ws128 = _sc_gather(x, order, w128)
    out_sorted = _moe_call(tmap, gmap, smap, emap, xs, ws128,
                           w1, v1, w2, gate_w, up_w, down_w)
    out = _sc_scatter(out_sorted, order)
    return out.reshape(hidden_states.shape)


# speedup vs baseline: 1.1653x; 1.1653x over previous
import functools

import jax
import jax.numpy as jnp
from jax import lax
from jax.experimental import pallas as pl
from jax.experimental.pallas import tpu as pltpu
from jax.experimental.pallas import tpu_sc as plsc

_T = 2048
_H = 1024
_F = 4096
_E = 16
_B = 256
_NT = _T // _B
_NBR = _NT + _E
_NSTEP = _NBR + _NT
_FC = 1024
_NF = _F // _FC
_NW = 32
_BPW = _T // _NW


def _sigmoid(x):
    return 1.0 / (1.0 + jnp.exp(-x))


def _router_body(x_ref, wr_ref, idx_ref, w_ref):
    logits = lax.dot_general(x_ref[...], wr_ref[...], (((1,), (1,)), ((), ())),
                             preferred_element_type=jnp.float32)
    top = jnp.max(logits, axis=1, keepdims=True)
    idx_ref[...] = jnp.argmax(logits, axis=1).astype(jnp.int32)[:, None]
    w_ref[...] = _sigmoid(top)


def _router(x, Wr):
    return pl.pallas_call(
        _router_body,
        out_shape=(jax.ShapeDtypeStruct((_T, 1), jnp.int32),
                   jax.ShapeDtypeStruct((_T, 1), jnp.float32)),
    )(x, Wr)


def _route_metadata(idx):
    counts = jnp.zeros((_E,), jnp.int32).at[idx].add(1)
    ends_c = jnp.cumsum(counts)
    offs = ends_c - counts
    tile_lo = offs // _B
    tile_hi = (ends_c + _B - 1) // _B
    nt = jnp.where(counts > 0, tile_hi - tile_lo, 0)
    cum_nt = jnp.cumsum(nt)
    total = cum_nt[_E - 1]
    s = jnp.arange(_NBR, dtype=jnp.int32)
    g = jnp.searchsorted(cum_nt, s, side="right").astype(jnp.int32)
    g = jnp.minimum(g, _E - 1)
    prev = jnp.where(g > 0, cum_nt[jnp.maximum(g - 1, 0)], 0)
    k = s - prev
    tile = tile_lo[g] + k
    start = jnp.maximum(offs[g], tile * _B)
    end = jnp.minimum(ends_c[g], (tile + 1) * _B)
    valid = s < total
    last = jnp.maximum(total - 1, 0)
    g = jnp.where(valid, g, g[last])
    tile = jnp.clip(jnp.where(valid, tile, tile[last]), 0, _NT - 1)
    start = jnp.where(valid, start, 0)
    end = jnp.where(valid, end, 0)
    stile = jnp.arange(_NT, dtype=jnp.int32)
    tmap = jnp.concatenate([tile, stile]).astype(jnp.int32)
    gmap = jnp.concatenate([g, jnp.zeros((_NT,), jnp.int32)]).astype(jnp.int32)
    smap = jnp.concatenate([start, stile * _B]).astype(jnp.int32)
    emap = jnp.concatenate([end, (stile + 1) * _B]).astype(jnp.int32)
    return tmap, gmap, smap, emap


def _gather_body(x_hbm, order_hbm, w_hbm, xs_hbm, ws_hbm,
                 idx_v, rows_v, wsv_v, sem, sem2):
    wid = lax.axis_index("s") * 2 + lax.axis_index("c")
    base = wid * _BPW
    pltpu.sync_copy(order_hbm.at[pl.ds(base, _BPW)], idx_v)
    cp_x = pltpu.async_copy(x_hbm.at[idx_v], rows_v, sem)
    cp_w = pltpu.async_copy(w_hbm.at[idx_v], wsv_v, sem2)
    cp_x.wait()
    cp_w.wait()
    pltpu.sync_copy(rows_v, xs_hbm.at[pl.ds(base, _BPW)])
    pltpu.sync_copy(wsv_v, ws_hbm.at[pl.ds(base, _BPW)])


def _sc_gather(x, order, w128):
    return pl.kernel(
        _gather_body,
        out_type=(jax.ShapeDtypeStruct((_T, _H), jnp.float32),
                  jax.ShapeDtypeStruct((_T, 128), jnp.float32)),
        mesh=plsc.VectorSubcoreMesh(core_axis_name="c", subcore_axis_name="s"),
        scratch_types=[pltpu.VMEM((_BPW,), jnp.int32),
                       pltpu.VMEM((_BPW, _H), jnp.float32),
                       pltpu.VMEM((_BPW, 128), jnp.float32),
                       pltpu.SemaphoreType.DMA,
                       pltpu.SemaphoreType.DMA],
    )(x, order, w128)


def _scatter_body(osort_hbm, order_hbm, out_hbm, idx_v, rows_v, sem):
    wid = lax.axis_index("s") * 2 + lax.axis_index("c")
    base = wid * _BPW
    pltpu.sync_copy(order_hbm.at[pl.ds(base, _BPW)], idx_v)
    pltpu.sync_copy(osort_hbm.at[pl.ds(base, _BPW)], rows_v)
    pltpu.async_copy(rows_v, out_hbm.at[idx_v], sem).wait()


def _sc_scatter(osort, order):
    return pl.kernel(
        _scatter_body,
        out_type=jax.ShapeDtypeStruct((_T, _H), jnp.float32),
        mesh=plsc.VectorSubcoreMesh(core_axis_name="c", subcore_axis_name="s"),
        scratch_types=[pltpu.VMEM((_BPW,), jnp.int32),
                       pltpu.VMEM((_BPW, _H), jnp.float32),
                       pltpu.SemaphoreType.DMA],
    )(osort, order)


def _moe_body(tmap, gmap, smap, emap, xs_ref, ws_ref,
              w1_ref, v1_ref, w2_ref, out_ref):
    f = pl.program_id(0)
    j = pl.program_id(1)

    @pl.when((f == 0) & (j == 0))
    def _init():
        out_ref[...] = jnp.zeros_like(out_ref)

    t = tmap[j]
    st = smap[j]
    en = emap[j]

    @pl.when(st < en)
    def _routed():
        xq = xs_ref[pl.ds(t * _B, _B), :].astype(jnp.bfloat16)
        a = lax.dot_general(xq, w1_ref[0].astype(jnp.bfloat16),
                            (((1,), (1,)), ((), ())),
                            preferred_element_type=jnp.float32)
        b = lax.dot_general(xq, v1_ref[0].astype(jnp.bfloat16),
                            (((1,), (1,)), ((), ())),
                            preferred_element_type=jnp.float32)
        h = a * _sigmoid(a) * b
        rows = t * _B + lax.broadcasted_iota(jnp.int32, (_B, 1), 0)
        h = jnp.where((rows >= st) & (rows < en), h * ws_ref[:, 0:1], 0.0)
        out_ref[pl.ds(t * _B, _B), :] += lax.dot_general(
            h.astype(jnp.bfloat16), w2_ref[0].astype(jnp.bfloat16),
            (((1,), (0,)), ((), ())),
            preferred_element_type=jnp.float32)


def _moe_grid_spec():
    return pltpu.PrefetchScalarGridSpec(
        num_scalar_prefetch=4,
        grid=(_NF, _NBR),
        in_specs=[
            pl.BlockSpec((_T, _H), lambda f, j, tm, gm, sm, em: (0, 0)),
            pl.BlockSpec((_B, 128), lambda f, j, tm, gm, sm, em: (tm[j], 0)),
            pl.BlockSpec((1, _FC, _H), lambda f, j, tm, gm, sm, em: (gm[j], f, 0)),
            pl.BlockSpec((1, _FC, _H), lambda f, j, tm, gm, sm, em: (gm[j], f, 0)),
            pl.BlockSpec((1, _FC, _H), lambda f, j, tm, gm, sm, em: (gm[j], f, 0)),
        ],
        out_specs=pl.BlockSpec((_T, _H), lambda f, j, tm, gm, sm, em: (0, 0)),
    )


def _moe_call(tmap, gmap, smap, emap, xs, ws, w1, v1, w2):
    return pl.pallas_call(
        _moe_body,
        grid_spec=_moe_grid_spec(),
        out_shape=jax.ShapeDtypeStruct((_T, _H), jnp.float32),
        compiler_params=pltpu.CompilerParams(
            dimension_semantics=("arbitrary", "arbitrary")),
    )(tmap, gmap, smap, emap, xs, ws, w1, v1, w2)


def _shared_body(xs_ref, prev_ref, gw_ref, uw_ref, dw_ref, out_ref):
    f = pl.program_id(0)
    t = pl.program_id(1)
    xq = xs_ref[pl.ds(t * _B, _B), :].astype(jnp.bfloat16)
    a = lax.dot_general(xq, gw_ref[...].astype(jnp.bfloat16),
                        (((1,), (1,)), ((), ())),
                        preferred_element_type=jnp.float32)
    b = lax.dot_general(xq, uw_ref[...].astype(jnp.bfloat16),
                        (((1,), (1,)), ((), ())),
                        preferred_element_type=jnp.float32)
    h = a * _sigmoid(a) * b
    contrib = lax.dot_general(
        h.astype(jnp.bfloat16), dw_ref[...].astype(jnp.bfloat16),
        (((1,), (1,)), ((), ())), preferred_element_type=jnp.float32)

    @pl.when(f == 0)
    def _first():
        out_ref[pl.ds(t * _B, _B), :] = prev_ref[pl.ds(t * _B, _B), :] + contrib

    @pl.when(f > 0)
    def _rest():
        out_ref[pl.ds(t * _B, _B), :] += contrib


def _shared_grid_spec():
    return pl.GridSpec(
        grid=(_NF, _NT),
        in_specs=[
            pl.BlockSpec((_T, _H), lambda f, t: (0, 0)),
            pl.BlockSpec((_T, _H), lambda f, t: (0, 0)),
            pl.BlockSpec((_FC, _H), lambda f, t: (f, 0)),
            pl.BlockSpec((_FC, _H), lambda f, t: (f, 0)),
            pl.BlockSpec((_H, _FC), lambda f, t: (0, f)),
        ],
        out_specs=pl.BlockSpec((_T, _H), lambda f, t: (0, 0)),
    )


def _shared_call(xs, prev, gate_w, up_w, down_w):
    return pl.pallas_call(
        _shared_body,
        grid_spec=_shared_grid_spec(),
        out_shape=jax.ShapeDtypeStruct((_T, _H), jnp.float32),
        input_output_aliases={1: 0},
        compiler_params=pltpu.CompilerParams(
            dimension_semantics=("arbitrary", "arbitrary")),
    )(xs, prev, gate_w, up_w, down_w)


def kernel(hidden_states, Wr, w1, v1, w2, gate_w, up_w, down_w):
    x = hidden_states.reshape(_T, _H)
    idx2, w2d = _router(x, Wr)
    idx = idx2[:, 0]
    order = jnp.argsort(idx, stable=True).astype(jnp.int32)
    tmap, gmap, smap, emap = _route_metadata(idx)
    w128 = jnp.broadcast_to(w2d, (_T, 128))
    xs, ws128 = _sc_gather(x, order, w128)
    routed = _moe_call(tmap, gmap, smap, emap, xs, ws128, w1, v1, w2)
    out_sorted = _shared_call(xs, routed, gate_w, up_w, down_w)
    out = _sc_scatter(out_sorted, order)
    return out.reshape(hidden_states.shape)
